# baseline (speedup 1.0000x reference)
import functools

import jax
import jax.numpy as jnp
from jax import lax
from jax.experimental import pallas as pl
from jax.experimental.pallas import tpu as pltpu
from jax.experimental.pallas import tpu_sc as plsc

_NC = 2
_NS = 16
_NW = _NC * _NS
_BLK = 128
_SCHUNK = 8
_NSETS = 3


def _ceil_to(x, m):
    return (x + m - 1) // m * m


def _mesh():
    return plsc.VectorSubcoreMesh(core_axis_name="c", subcore_axis_name="s")


_SC_PARAMS = pltpu.CompilerParams(use_tc_tiling_on_sc=False)


def _make_deg_kernel(n_pad, ebm, ebx):
    ns = _NSETS
    bpt = (ebm + ebx) // _NW
    wpt = bpt // _SCHUNK
    rpt = n_pad // _NS

    def body(main_hbm, extra_hbm, zeros_hbm, out_hbm, acc, idx, ones, zb,
             semS, semI):
        cid = lax.axis_index("c")
        sid = lax.axis_index("s")
        wid = cid * _NS + sid
        base = wid * bpt

        def load_idx(blk, buf, sem):
            @pl.when(blk < ebm)
            def _():
                pltpu.async_copy(main_hbm.at[1, pl.ds(blk, _SCHUNK)], buf,
                                 sem)

            @pl.when(blk >= ebm)
            def _():
                pltpu.async_copy(extra_hbm.at[pl.ds(blk - ebm, _SCHUNK)],
                                 buf, sem)

        for i in range(_BLK // 16):
            ones[pl.ds(i * 16, 16)] = jnp.full((16,), 1.0, jnp.float32)

        pltpu.sync_copy(zeros_hbm.at[pl.ds(sid * rpt, rpt)], zb)
        pltpu.sync_copy(zb, acc.at[pl.ds(sid * rpt, rpt)])
        plsc.subcore_barrier()

        load_idx(base, idx.at[0], semI)
        pltpu.make_async_copy(main_hbm.at[0, pl.ds(0, _SCHUNK)],
                              idx.at[0], semI).wait()
        if wpt > 1:
            load_idx(base + _SCHUNK, idx.at[1], semI)

        def wave(w, _):
            prv = (w + ns - 1) % ns
            cur = w % ns
            nxt = (w + 1) % ns
            nn = (w + 2) % ns

            @pl.when(w > 0)
            def _():
                pltpu.make_async_copy(main_hbm.at[0, pl.ds(0, _SCHUNK)],
                                      idx.at[prv], semS).wait()

            for j in range(_SCHUNK):
                pltpu.async_copy(ones, acc.at[idx.at[cur, j]], semS,
                                 add=True)

            @pl.when(w < wpt - 1)
            def _():
                pltpu.make_async_copy(main_hbm.at[0, pl.ds(0, _SCHUNK)],
                                      idx.at[nxt], semI).wait()

            @pl.when(w < wpt - 2)
            def _():
                load_idx(base + (w + 2) * _SCHUNK, idx.at[nn], semI)

            return ()

        lax.fori_loop(0, wpt, wave, (), unroll=False)

        pltpu.make_async_copy(main_hbm.at[0, pl.ds(0, _SCHUNK)],
                              idx.at[(wpt - 1) % ns], semS).wait()
        plsc.subcore_barrier()
        pltpu.sync_copy(acc.at[pl.ds(sid * rpt, rpt)], zb)
        pltpu.sync_copy(zb, out_hbm.at[pl.ds(cid * n_pad + sid * rpt, rpt)])

    return pl.kernel(
        body,
        out_type=jax.ShapeDtypeStruct((_NC * n_pad,), jnp.float32),
        mesh=_mesh(),
        compiler_params=_SC_PARAMS,
        scratch_types=[
            pltpu.VMEM_SHARED((n_pad,), jnp.float32),
            pltpu.VMEM((_NSETS, _SCHUNK, _BLK), jnp.int32),
            pltpu.VMEM((_BLK,), jnp.float32),
            pltpu.VMEM((n_pad // _NS,), jnp.float32),
            pltpu.SemaphoreType.DMA,
            pltpu.SemaphoreType.DMA,
        ],
    )


def _make_agg_kernel(n_pad, ebm, ebx, width):
    schunk = _SCHUNK if width <= 8 else _SCHUNK // 2
    ns = _NSETS
    bpt = (ebm + ebx) // _NW
    wpt = bpt // schunk
    rpt = n_pad // _NS
    stage = ns * schunk * _BLK

    def body(g_hbm, main_hbm, xsrc_hbm, xdst_hbm, zeros_hbm, dummy_hbm,
             out_hbm, acc, sidx, didx, rows, semG, semS, semI):
        cid = lax.axis_index("c")
        sid = lax.axis_index("s")
        wid = cid * _NS + sid
        base = wid * bpt

        def load_idx2(blk, sbuf, dbuf, sem):
            @pl.when(blk < ebm)
            def _():
                pltpu.async_copy(main_hbm.at[0, pl.ds(blk, schunk)], sbuf,
                                 sem)
                pltpu.async_copy(main_hbm.at[1, pl.ds(blk, schunk)], dbuf,
                                 sem)

            @pl.when(blk >= ebm)
            def _():
                pltpu.async_copy(xsrc_hbm.at[pl.ds(blk - ebm, schunk)],
                                 sbuf, sem)
                pltpu.async_copy(xdst_hbm.at[pl.ds(blk - ebm, schunk)],
                                 dbuf, sem)

        def rowbuf(b):
            return rows.at[pl.ds(b * _BLK, _BLK)]

        def rowset(s):
            return rows.at[pl.ds(s * schunk * _BLK, schunk * _BLK)]

        pos = 0
        while pos < rpt:
            sz = min(stage, rpt - pos)
            pltpu.sync_copy(zeros_hbm.at[pl.ds(sid * rpt + pos, sz)],
                            rows.at[pl.ds(0, sz)])
            pltpu.sync_copy(rows.at[pl.ds(0, sz)],
                            acc.at[pl.ds(sid * rpt + pos, sz)])
            pos += sz
        plsc.subcore_barrier()

        load_idx2(base, sidx.at[0], didx.at[0], semI)
        pltpu.make_async_copy(main_hbm.at[0, pl.ds(0, schunk)], sidx.at[0],
                              semI).wait()
        pltpu.make_async_copy(main_hbm.at[0, pl.ds(0, schunk)], didx.at[0],
                              semI).wait()
        if wpt > 1:
            load_idx2(base + schunk, sidx.at[1], didx.at[1], semI)
        for j in range(schunk):
            pltpu.async_copy(g_hbm.at[sidx.at[0, j]], rowbuf(j), semG)

        def wave(w, _):
            prv = (w + ns - 1) % ns
            cur = w % ns
            nxt = (w + 1) % ns
            nn = (w + 2) % ns

            @pl.when(w > 0)
            def _():
                pltpu.make_async_copy(dummy_hbm, rowset(prv), semS).wait()

            pltpu.make_async_copy(dummy_hbm, rowset(cur), semG).wait()
            for j in range(schunk):
                pltpu.async_copy(rowbuf(cur * schunk + j),
                                 acc.at[didx.at[cur, j]], semS, add=True)

            @pl.when(w < wpt - 1)
            def _():
                pltpu.make_async_copy(main_hbm.at[0, pl.ds(0, schunk)],
                                      sidx.at[nxt], semI).wait()
                pltpu.make_async_copy(main_hbm.at[0, pl.ds(0, schunk)],
                                      didx.at[nxt], semI).wait()

            @pl.when(w < wpt - 2)
            def _():
                load_idx2(base + (w + 2) * schunk, sidx.at[nn],
                          didx.at[nn], semI)

            @pl.when(w < wpt - 1)
            def _():
                for j in range(schunk):
                    pltpu.async_copy(g_hbm.at[sidx.at[nxt, j]],
                                     rowbuf(nxt * schunk + j), semG)

            return ()

        lax.fori_loop(0, wpt, wave, (), unroll=False)

        pltpu.make_async_copy(dummy_hbm, rowset((wpt - 1) % ns),
                              semS).wait()
        plsc.subcore_barrier()
        pos = 0
        while pos < rpt:
            sz = min(stage, rpt - pos)
            pltpu.sync_copy(acc.at[pl.ds(sid * rpt + pos, sz)],
                            rows.at[pl.ds(0, sz)])
            pltpu.sync_copy(rows.at[pl.ds(0, sz)],
                            out_hbm.at[pl.ds(cid * n_pad + sid * rpt + pos,
                                             sz)])
            pos += sz

    return pl.kernel(
        body,
        out_type=jax.ShapeDtypeStruct((_NC * n_pad, width), jnp.float32),
        mesh=_mesh(),
        compiler_params=_SC_PARAMS,
        scratch_types=[
            pltpu.VMEM_SHARED((n_pad, width), jnp.float32),
            pltpu.VMEM((ns, schunk, _BLK), jnp.int32),
            pltpu.VMEM((ns, schunk, _BLK), jnp.int32),
            pltpu.VMEM((ns * schunk * _BLK, width), jnp.float32),
            pltpu.SemaphoreType.DMA,
            pltpu.SemaphoreType.DMA,
            pltpu.SemaphoreType.DMA,
        ],
    )


def _lin1_body(xw_ref, p0_ref, p1_ref, w1bd_ref, e8_ref, g_ref):
    dis = lax.rsqrt(jnp.maximum(p0_ref[...] + p1_ref[...], 1.0))
    disg = jnp.dot(dis, e8_ref[...], preferred_element_type=jnp.float32)
    h = jnp.dot(xw_ref[...], w1bd_ref[...],
                preferred_element_type=jnp.float32)
    g_ref[...] = h * disg


def _lin2_body(p0w_ref, p1w_ref, p0d_ref, p1d_ref, w2bd_ref, e8a_ref,
               e8b_ref, b1g_ref, g2_ref):
    dis = lax.rsqrt(jnp.maximum(p0d_ref[...] + p1d_ref[...], 1.0))
    disg = jnp.dot(dis, e8a_ref[...], preferred_element_type=jnp.float32)
    s = p0w_ref[...] + p1w_ref[...]
    f = jnp.maximum(s * disg + b1g_ref[...], 0.0)
    z = jnp.dot(f, w2bd_ref[...], preferred_element_type=jnp.float32)
    g2_ref[...] = z * jnp.dot(dis, e8b_ref[...],
                              preferred_element_type=jnp.float32)


def _out_body(p0w_ref, p1w_ref, p0d_ref, p1d_ref, e16_ref, gsum_ref,
              swp_ref, b2g_ref, sel_ref, o_ref):
    dis = lax.rsqrt(jnp.maximum(p0d_ref[...] + p1d_ref[...], 1.0))
    disg = jnp.dot(dis, e16_ref[...], preferred_element_type=jnp.float32)
    o = (p0w_ref[...] + p1w_ref[...]) * disg + b2g_ref[...]
    m = jnp.maximum(o, jnp.dot(o, swp_ref[...],
                               preferred_element_type=jnp.float32))
    e = jnp.exp(o - m) * sel_ref[...]
    lse = jnp.log(jnp.dot(e, gsum_ref[...],
                          preferred_element_type=jnp.float32))
    o_ref[...] = o - m - lse


def kernel(x, edge_index, W1, b1, W2, b2):
    n, d_in = x.shape
    e = edge_index.shape[1]
    d_hid = W1.shape[1]
    d_out = W2.shape[1]
    f32 = jnp.float32

    n_pad = _ceil_to(n + 1, 2048)
    rw1 = n_pad * d_hid // 128
    rw2 = n_pad * 8 // 128
    rdeg = n_pad // 8

    em = e - e % (_BLK * _SCHUNK)
    ep = _ceil_to(e + n, _NW * _SCHUNK * _BLK)
    npad_e = ep - (e + n)
    ebm = em // _BLK
    ebx = (ep - em) // _BLK
    loop = jnp.arange(n, dtype=jnp.int32)
    pad_src = jnp.arange(npad_e, dtype=jnp.int32) % 1024
    pad_dst = n + jnp.arange(npad_e, dtype=jnp.int32) % (n_pad - n)
    main = edge_index[:, :em].reshape(2, ebm, _BLK)
    xsrc = jnp.concatenate([edge_index[0, em:], loop, pad_src]).reshape(
        ebx, _BLK)
    xdst = jnp.concatenate([edge_index[1, em:], loop, pad_dst]).reshape(
        ebx, _BLK)

    zeros1 = jnp.zeros((n_pad,), f32)
    zeros_h = jnp.zeros((n_pad, d_hid), f32)
    zeros_o = jnp.zeros((n_pad, 8), f32)
    dummy_h = jnp.zeros((_SCHUNK // 2 * _BLK, d_hid), f32)
    dummy_o = jnp.zeros((_SCHUNK * _BLK, 8), f32)

    W1bd = jax.scipy.linalg.block_diag(*([W1] * 8))
    W2p = jnp.concatenate([W2, jnp.zeros((d_hid, 8 - d_out), f32)], axis=1)
    W2bd = jax.scipy.linalg.block_diag(*([W2p] * 8))
    E8_128 = jnp.repeat(jnp.eye(8, dtype=f32), d_hid, axis=1)
    E8_64 = jnp.repeat(jnp.eye(8, dtype=f32), 8, axis=1)
    E16_128 = jnp.repeat(jnp.eye(16, dtype=f32), 8, axis=1)
    b1g = jnp.tile(b1, 8).reshape(1, 128)
    b2g = jnp.tile(jnp.concatenate([b2, jnp.zeros((8 - d_out,), f32)]),
                   16).reshape(1, 128)
    col = jnp.arange(128)
    sel = (col % 8 < d_out).astype(f32).reshape(1, 128)
    gsum = ((col[:, None] // 8 == col[None, :] // 8)
            & (col[:, None] % 8 < d_out)).astype(f32)
    swap_idx = jnp.where(col % 8 < d_out, col ^ 1, col)
    swp = (col[:, None] == swap_idx[None, :]).astype(f32)

    degp = _make_deg_kernel(n_pad, ebm, ebx)(main, xdst, zeros1)
    p0d8 = degp.reshape(2 * rdeg, 8)
    p0d16 = degp.reshape(rdeg, 16)

    xw = x.reshape(n // 8, 8 * d_in)

    bw = n_pad // 8 // 8
    bf = n_pad // 16 // 8

    g1w = pl.pallas_call(
        _lin1_body,
        grid=(8,),
        in_specs=[
            pl.BlockSpec((bw, 8 * d_in), lambda i: (i, 0)),
            pl.BlockSpec((bw, 8), lambda i: (i, 0)),
            pl.BlockSpec((bw, 8), lambda i: (i + 8, 0)),
            pl.BlockSpec((8 * d_in, 128), lambda i: (0, 0)),
            pl.BlockSpec((8, 128), lambda i: (0, 0)),
        ],
        out_specs=pl.BlockSpec((bw, 128), lambda i: (i, 0)),
        out_shape=jax.ShapeDtypeStruct((n_pad // 8, 128), f32),
    )(xw, p0d8, p0d8, W1bd, E8_128)
    g1 = g1w.reshape(n_pad, d_hid)

    aggp1 = _make_agg_kernel(n_pad, ebm, ebx, d_hid)(
        g1, main, xsrc, xdst, zeros_h, dummy_h)
    ap1w = aggp1.reshape(2 * n_pad * d_hid // 128, 128)

    g2w = pl.pallas_call(
        _lin2_body,
        grid=(8,),
        in_specs=[
            pl.BlockSpec((bw, 128), lambda i: (i, 0)),
            pl.BlockSpec((bw, 128), lambda i: (i + 8, 0)),
            pl.BlockSpec((bw, 8), lambda i: (i, 0)),
            pl.BlockSpec((bw, 8), lambda i: (i + 8, 0)),
            pl.BlockSpec((128, 64), lambda i: (0, 0)),
            pl.BlockSpec((8, 128), lambda i: (0, 0)),
            pl.BlockSpec((8, 64), lambda i: (0, 0)),
            pl.BlockSpec((1, 128), lambda i: (0, 0)),
        ],
        out_specs=pl.BlockSpec((bw, 64), lambda i: (i, 0)),
        out_shape=jax.ShapeDtypeStruct((n_pad // 8, 64), f32),
    )(ap1w, ap1w, p0d8, p0d8, W2bd, E8_128, E8_64, b1g)
    g2 = g2w.reshape(n_pad, 8)

    aggp2 = _make_agg_kernel(n_pad, ebm, ebx, 8)(
        g2, main, xsrc, xdst, zeros_o, dummy_o)
    ap2w = aggp2.reshape(2 * n_pad * 8 // 128, 128)

    outw = pl.pallas_call(
        _out_body,
        grid=(8,),
        in_specs=[
            pl.BlockSpec((bf, 128), lambda i: (i, 0)),
            pl.BlockSpec((bf, 128), lambda i: (i + 8, 0)),
            pl.BlockSpec((bf, 16), lambda i: (i, 0)),
            pl.BlockSpec((bf, 16), lambda i: (i + 8, 0)),
            pl.BlockSpec((16, 128), lambda i: (0, 0)),
            pl.BlockSpec((128, 128), lambda i: (0, 0)),
            pl.BlockSpec((128, 128), lambda i: (0, 0)),
            pl.BlockSpec((1, 128), lambda i: (0, 0)),
            pl.BlockSpec((1, 128), lambda i: (0, 0)),
        ],
        out_specs=pl.BlockSpec((bf, 128), lambda i: (i, 0)),
        out_shape=jax.ShapeDtypeStruct((n_pad // 16, 128), f32),
    )(ap2w, ap2w, p0d16, p0d16, E16_128, gsum, swp, b2g, sel)

    if d_out == 2:
        pairs = outw.reshape(n_pad * 4, 2)
        return lax.slice(pairs, (0, 0), (4 * (n - 1) + 1, 2), (4, 1))
    flat = outw.reshape(n_pad * 8)
    cols = [lax.slice(flat, (c,), (n * 8,), (8,)) for c in range(d_out)]
    return jnp.stack(cols, axis=1)


# speedup vs baseline: 1.6359x; 1.6359x over previous
import functools

import jax
import jax.numpy as jnp
from jax import lax
from jax.experimental import pallas as pl
from jax.experimental.pallas import tpu as pltpu
from jax.experimental.pallas import tpu_sc as plsc

_NC = 2
_NS = 16
_NW = _NC * _NS
_BLK = 128
_SCHUNK = 8
_NSETS = 3


def _ceil_to(x, m):
    return (x + m - 1) // m * m


def _mesh():
    return plsc.VectorSubcoreMesh(core_axis_name="c", subcore_axis_name="s")


_SC_PARAMS = pltpu.CompilerParams(use_tc_tiling_on_sc=False)


def _make_deg_kernel(n_pad, ebm, ebx):
    ns = _NSETS
    bpt = (ebm + ebx) // _NW
    wpt = bpt // _SCHUNK
    rpt = n_pad // _NS

    def body(main_hbm, extra_hbm, zeros_hbm, out_hbm, acc, idx, ones, zb,
             semS, semI):
        cid = lax.axis_index("c")
        sid = lax.axis_index("s")
        wid = cid * _NS + sid
        base = wid * bpt

        def load_idx(blk, buf, sem):
            @pl.when(blk < ebm)
            def _():
                pltpu.async_copy(main_hbm.at[1, pl.ds(blk, _SCHUNK)], buf,
                                 sem)

            @pl.when(blk >= ebm)
            def _():
                pltpu.async_copy(extra_hbm.at[pl.ds(blk - ebm, _SCHUNK)],
                                 buf, sem)

        for i in range(_BLK // 16):
            ones[pl.ds(i * 16, 16)] = jnp.full((16,), 1.0, jnp.float32)

        pltpu.sync_copy(zeros_hbm.at[pl.ds(sid * rpt, rpt)], zb)
        pltpu.sync_copy(zb, acc.at[pl.ds(sid * rpt, rpt)])
        plsc.subcore_barrier()

        load_idx(base, idx.at[0], semI)
        pltpu.make_async_copy(main_hbm.at[0, pl.ds(0, _SCHUNK)],
                              idx.at[0], semI).wait()
        if wpt > 1:
            load_idx(base + _SCHUNK, idx.at[1], semI)

        def wave(w, _):
            prv = (w + ns - 1) % ns
            cur = w % ns
            nxt = (w + 1) % ns
            nn = (w + 2) % ns

            @pl.when(w > 0)
            def _():
                pltpu.make_async_copy(main_hbm.at[0, pl.ds(0, _SCHUNK)],
                                      idx.at[prv], semS).wait()

            for j in range(_SCHUNK):
                pltpu.async_copy(ones, acc.at[idx.at[cur, j]], semS,
                                 add=True)

            @pl.when(w < wpt - 1)
            def _():
                pltpu.make_async_copy(main_hbm.at[0, pl.ds(0, _SCHUNK)],
                                      idx.at[nxt], semI).wait()

            @pl.when(w < wpt - 2)
            def _():
                load_idx(base + (w + 2) * _SCHUNK, idx.at[nn], semI)

            return ()

        lax.fori_loop(0, wpt, wave, (), unroll=False)

        pltpu.make_async_copy(main_hbm.at[0, pl.ds(0, _SCHUNK)],
                              idx.at[(wpt - 1) % ns], semS).wait()
        plsc.subcore_barrier()
        pltpu.sync_copy(acc.at[pl.ds(sid * rpt, rpt)], zb)
        pltpu.sync_copy(zb, out_hbm.at[pl.ds(cid * n_pad + sid * rpt, rpt)])

    return pl.kernel(
        body,
        out_type=jax.ShapeDtypeStruct((_NC * n_pad,), jnp.float32),
        mesh=_mesh(),
        compiler_params=_SC_PARAMS,
        scratch_types=[
            pltpu.VMEM_SHARED((n_pad,), jnp.float32),
            pltpu.VMEM((_NSETS, _SCHUNK, _BLK), jnp.int32),
            pltpu.VMEM((_BLK,), jnp.float32),
            pltpu.VMEM((n_pad // _NS,), jnp.float32),
            pltpu.SemaphoreType.DMA,
            pltpu.SemaphoreType.DMA,
        ],
    )


def _make_agg_kernel(n_pad, ebm, ebx, width):
    schunk = _SCHUNK if width <= 8 else _SCHUNK // 2
    ns = _NSETS
    bpt = (ebm + ebx) // _NW
    wpt = bpt // schunk
    rpt = n_pad // _NS
    stage = ns * schunk * _BLK

    def body(g_hbm, main_hbm, xsrc_hbm, xdst_hbm, zeros_hbm, dummy_hbm,
             out_hbm, acc, sidx, didx, rows, semG, semS, semI):
        cid = lax.axis_index("c")
        sid = lax.axis_index("s")
        wid = cid * _NS + sid
        base = wid * bpt

        def load_idx2(blk, sbuf, dbuf, sem):
            @pl.when(blk < ebm)
            def _():
                pltpu.async_copy(main_hbm.at[0, pl.ds(blk, schunk)], sbuf,
                                 sem)
                pltpu.async_copy(main_hbm.at[1, pl.ds(blk, schunk)], dbuf,
                                 sem)

            @pl.when(blk >= ebm)
            def _():
                pltpu.async_copy(xsrc_hbm.at[pl.ds(blk - ebm, schunk)],
                                 sbuf, sem)
                pltpu.async_copy(xdst_hbm.at[pl.ds(blk - ebm, schunk)],
                                 dbuf, sem)

        def rowbuf(b):
            return rows.at[pl.ds(b * _BLK, _BLK)]

        def rowset(s):
            return rows.at[pl.ds(s * schunk * _BLK, schunk * _BLK)]

        pos = 0
        while pos < rpt:
            sz = min(stage, rpt - pos)
            pltpu.sync_copy(zeros_hbm.at[pl.ds(sid * rpt + pos, sz)],
                            rows.at[pl.ds(0, sz)])
            pltpu.sync_copy(rows.at[pl.ds(0, sz)],
                            acc.at[pl.ds(sid * rpt + pos, sz)])
            pos += sz
        plsc.subcore_barrier()

        load_idx2(base, sidx.at[0], didx.at[0], semI)
        pltpu.make_async_copy(main_hbm.at[0, pl.ds(0, schunk)], sidx.at[0],
                              semI).wait()
        pltpu.make_async_copy(main_hbm.at[0, pl.ds(0, schunk)], didx.at[0],
                              semI).wait()
        if wpt > 1:
            load_idx2(base + schunk, sidx.at[1], didx.at[1], semI)
        for j in range(schunk):
            pltpu.async_copy(g_hbm.at[sidx.at[0, j]], rowbuf(j), semG)

        def wave(w, _):
            prv = (w + ns - 1) % ns
            cur = w % ns
            nxt = (w + 1) % ns
            nn = (w + 2) % ns

            @pl.when(w > 0)
            def _():
                pltpu.make_async_copy(dummy_hbm, rowset(prv), semS).wait()

            pltpu.make_async_copy(dummy_hbm, rowset(cur), semG).wait()
            for j in range(schunk):
                pltpu.async_copy(rowbuf(cur * schunk + j),
                                 acc.at[didx.at[cur, j]], semS, add=True)

            @pl.when(w < wpt - 1)
            def _():
                pltpu.make_async_copy(main_hbm.at[0, pl.ds(0, schunk)],
                                      sidx.at[nxt], semI).wait()
                pltpu.make_async_copy(main_hbm.at[0, pl.ds(0, schunk)],
                                      didx.at[nxt], semI).wait()

            @pl.when(w < wpt - 2)
            def _():
                load_idx2(base + (w + 2) * schunk, sidx.at[nn],
                          didx.at[nn], semI)

            @pl.when(w < wpt - 1)
            def _():
                for j in range(schunk):
                    pltpu.async_copy(g_hbm.at[sidx.at[nxt, j]],
                                     rowbuf(nxt * schunk + j), semG)

            return ()

        lax.fori_loop(0, wpt, wave, (), unroll=False)

        pltpu.make_async_copy(dummy_hbm, rowset((wpt - 1) % ns),
                              semS).wait()
        plsc.subcore_barrier()
        pos = 0
        while pos < rpt:
            sz = min(stage, rpt - pos)
            pltpu.sync_copy(acc.at[pl.ds(sid * rpt + pos, sz)],
                            rows.at[pl.ds(0, sz)])
            pltpu.sync_copy(rows.at[pl.ds(0, sz)],
                            out_hbm.at[pl.ds(cid * n_pad + sid * rpt + pos,
                                             sz)])
            pos += sz

    return pl.kernel(
        body,
        out_type=jax.ShapeDtypeStruct((_NC * n_pad, width), jnp.float32),
        mesh=_mesh(),
        compiler_params=_SC_PARAMS,
        scratch_types=[
            pltpu.VMEM_SHARED((n_pad, width), jnp.float32),
            pltpu.VMEM((ns, schunk, _BLK), jnp.int32),
            pltpu.VMEM((ns, schunk, _BLK), jnp.int32),
            pltpu.VMEM((ns * schunk * _BLK, width), jnp.float32),
            pltpu.SemaphoreType.DMA,
            pltpu.SemaphoreType.DMA,
            pltpu.SemaphoreType.DMA,
        ],
    )


def _lin1_body(xw_ref, p0_ref, p1_ref, w1bd_ref, e8_ref, g_ref):
    dis = lax.rsqrt(jnp.maximum(p0_ref[...] + p1_ref[...], 1.0))
    disg = jnp.dot(dis, e8_ref[...], preferred_element_type=jnp.float32)
    h = jnp.dot(xw_ref[...], w1bd_ref[...],
                preferred_element_type=jnp.float32)
    g_ref[...] = h * disg


def _lin2_body(p0w_ref, p1w_ref, p0d_ref, p1d_ref, w2bd_ref, e8a_ref,
               e8b_ref, b1g_ref, g2_ref):
    dis = lax.rsqrt(jnp.maximum(p0d_ref[...] + p1d_ref[...], 1.0))
    disg = jnp.dot(dis, e8a_ref[...], preferred_element_type=jnp.float32)
    s = p0w_ref[...] + p1w_ref[...]
    f = jnp.maximum(s * disg + b1g_ref[...], 0.0)
    z = jnp.dot(f, w2bd_ref[...], preferred_element_type=jnp.float32)
    g2_ref[...] = z * jnp.dot(dis, e8b_ref[...],
                              preferred_element_type=jnp.float32)


def _out_body(p0w_ref, p1w_ref, p0d_ref, p1d_ref, e16_ref, gsum_ref,
              swp_ref, b2g_ref, sel_ref, o_ref):
    dis = lax.rsqrt(jnp.maximum(p0d_ref[...] + p1d_ref[...], 1.0))
    disg = jnp.dot(dis, e16_ref[...], preferred_element_type=jnp.float32)
    o = (p0w_ref[...] + p1w_ref[...]) * disg + b2g_ref[...]
    m = jnp.maximum(o, jnp.dot(o, swp_ref[...],
                               preferred_element_type=jnp.float32))
    e = jnp.exp(o - m) * sel_ref[...]
    lse = jnp.log(jnp.dot(e, gsum_ref[...],
                          preferred_element_type=jnp.float32))
    o_ref[...] = o - m - lse


def kernel(x, edge_index, W1, b1, W2, b2):
    n, d_in = x.shape
    e = edge_index.shape[1]
    d_hid = W1.shape[1]
    d_out = W2.shape[1]
    f32 = jnp.float32

    n_pad = _ceil_to(n + 1, 2048)
    rw1 = n_pad * d_hid // 128
    rw2 = n_pad * 8 // 128
    rdeg = n_pad // 8

    em = e - e % (_BLK * _SCHUNK)
    ep = _ceil_to(e + n, _NW * _SCHUNK * _BLK)
    npad_e = ep - (e + n)
    ebm = em // _BLK
    ebx = (ep - em) // _BLK
    loop = jnp.arange(n, dtype=jnp.int32)
    pad_src = jnp.arange(npad_e, dtype=jnp.int32) % 1024
    pad_dst = n + jnp.arange(npad_e, dtype=jnp.int32) % (n_pad - n)
    main = edge_index[:, :em].reshape(2, ebm, _BLK)
    xsrc = jnp.concatenate([edge_index[0, em:], loop, pad_src]).reshape(
        ebx, _BLK)
    xdst = jnp.concatenate([edge_index[1, em:], loop, pad_dst]).reshape(
        ebx, _BLK)

    zeros1 = jnp.zeros((n_pad,), f32)
    zeros_h = jnp.zeros((n_pad, d_hid), f32)
    zeros_o = jnp.zeros((n_pad, 8), f32)
    dummy_h = jnp.zeros((_SCHUNK // 2 * _BLK, d_hid), f32)
    dummy_o = jnp.zeros((_SCHUNK * _BLK, 8), f32)

    W1bd = jax.scipy.linalg.block_diag(*([W1] * 8))
    W2p = jnp.concatenate([W2, jnp.zeros((d_hid, 8 - d_out), f32)], axis=1)
    W2bd = jax.scipy.linalg.block_diag(*([W2p] * 8))
    E8_128 = jnp.repeat(jnp.eye(8, dtype=f32), d_hid, axis=1)
    E8_64 = jnp.repeat(jnp.eye(8, dtype=f32), 8, axis=1)
    E16_128 = jnp.repeat(jnp.eye(16, dtype=f32), 8, axis=1)
    b1g = jnp.tile(b1, 8).reshape(1, 128)
    b2g = jnp.tile(jnp.concatenate([b2, jnp.zeros((8 - d_out,), f32)]),
                   16).reshape(1, 128)
    col = jnp.arange(128)
    sel = (col % 8 < d_out).astype(f32).reshape(1, 128)
    gsum = ((col[:, None] // 8 == col[None, :] // 8)
            & (col[:, None] % 8 < d_out)).astype(f32)
    swap_idx = jnp.where(col % 8 < d_out, col ^ 1, col)
    swp = (col[:, None] == swap_idx[None, :]).astype(f32)

    degp = _make_deg_kernel(n_pad, ebm, ebx)(main, xdst, zeros1)
    p0d8 = degp.reshape(2 * rdeg, 8)
    p0d16 = degp.reshape(rdeg, 16)

    xw = jnp.pad(x.reshape(n // 8, 8 * d_in),
                 ((0, (n_pad - n) // 8), (0, 0)))

    bw = n_pad // 8 // 8
    bf = n_pad // 16 // 8

    g1w = pl.pallas_call(
        _lin1_body,
        grid=(8,),
        in_specs=[
            pl.BlockSpec((bw, 8 * d_in), lambda i: (i, 0)),
            pl.BlockSpec((bw, 8), lambda i: (i, 0)),
            pl.BlockSpec((bw, 8), lambda i: (i + 8, 0)),
            pl.BlockSpec((8 * d_in, 128), lambda i: (0, 0)),
            pl.BlockSpec((8, 128), lambda i: (0, 0)),
        ],
        out_specs=pl.BlockSpec((bw, 128), lambda i: (i, 0)),
        out_shape=jax.ShapeDtypeStruct((n_pad // 8, 128), f32),
    )(xw, p0d8, p0d8, W1bd, E8_128)
    g1 = g1w.reshape(n_pad, d_hid)

    aggp1 = _make_agg_kernel(n_pad, ebm, ebx, d_hid)(
        g1, main, xsrc, xdst, zeros_h, dummy_h)
    ap1w = aggp1.reshape(2 * n_pad * d_hid // 128, 128)

    g2w = pl.pallas_call(
        _lin2_body,
        grid=(8,),
        in_specs=[
            pl.BlockSpec((bw, 128), lambda i: (i, 0)),
            pl.BlockSpec((bw, 128), lambda i: (i + 8, 0)),
            pl.BlockSpec((bw, 8), lambda i: (i, 0)),
            pl.BlockSpec((bw, 8), lambda i: (i + 8, 0)),
            pl.BlockSpec((128, 64), lambda i: (0, 0)),
            pl.BlockSpec((8, 128), lambda i: (0, 0)),
            pl.BlockSpec((8, 64), lambda i: (0, 0)),
            pl.BlockSpec((1, 128), lambda i: (0, 0)),
        ],
        out_specs=pl.BlockSpec((bw, 64), lambda i: (i, 0)),
        out_shape=jax.ShapeDtypeStruct((n_pad // 8, 64), f32),
    )(ap1w, ap1w, p0d8, p0d8, W2bd, E8_128, E8_64, b1g)
    g2 = g2w.reshape(n_pad, 8)

    aggp2 = _make_agg_kernel(n_pad, ebm, ebx, 8)(
        g2, main, xsrc, xdst, zeros_o, dummy_o)
    ap2w = aggp2.reshape(2 * n_pad * 8 // 128, 128)

    outw = pl.pallas_call(
        _out_body,
        grid=(8,),
        in_specs=[
            pl.BlockSpec((bf, 128), lambda i: (i, 0)),
            pl.BlockSpec((bf, 128), lambda i: (i + 8, 0)),
            pl.BlockSpec((bf, 16), lambda i: (i, 0)),
            pl.BlockSpec((bf, 16), lambda i: (i + 8, 0)),
            pl.BlockSpec((16, 128), lambda i: (0, 0)),
            pl.BlockSpec((128, 128), lambda i: (0, 0)),
            pl.BlockSpec((128, 128), lambda i: (0, 0)),
            pl.BlockSpec((1, 128), lambda i: (0, 0)),
            pl.BlockSpec((1, 128), lambda i: (0, 0)),
        ],
        out_specs=pl.BlockSpec((bf, 128), lambda i: (i, 0)),
        out_shape=jax.ShapeDtypeStruct((n_pad // 16, 128), f32),
    )(ap2w, ap2w, p0d16, p0d16, E16_128, gsum, swp, b2g, sel)

    flat = outw.reshape(n_pad * 8)
    cols = [lax.slice(flat, (c,), (n * 8,), (8,)) for c in range(d_out)]
    return jnp.stack(cols, axis=1)


# speedup vs baseline: 1.6365x; 1.0003x over previous
import jax
import jax.numpy as jnp
from jax import lax
from jax.experimental import pallas as pl
from jax.experimental.pallas import tpu as pltpu
from jax.experimental.pallas import tpu_sc as plsc

_NC = 2
_NS = 16
_NW = _NC * _NS
_BLK = 128
_SCHUNK = 8
_NSETS = 3


def _ceil_to(x, m):
    return (x + m - 1) // m * m


def _mesh():
    return plsc.VectorSubcoreMesh(core_axis_name="c", subcore_axis_name="s")


_SC_PARAMS = pltpu.CompilerParams(use_tc_tiling_on_sc=False)


def _make_deg_kernel(n_pad, ebm, ebx):
    ns = _NSETS
    bpt = (ebm + ebx) // _NW
    wpt = bpt // _SCHUNK
    rpt = n_pad // _NS

    def body(main_hbm, extra_hbm, zeros_hbm, out_hbm, acc, idx, ones, zb,
             semS, semI):
        cid = lax.axis_index("c")
        sid = lax.axis_index("s")
        wid = cid * _NS + sid
        base = wid * bpt

        def load_idx(blk, buf, sem):
            @pl.when(blk < ebm)
            def _():
                pltpu.async_copy(main_hbm.at[1, pl.ds(blk, _SCHUNK)], buf,
                                 sem)

            @pl.when(blk >= ebm)
            def _():
                pltpu.async_copy(extra_hbm.at[pl.ds(blk - ebm, _SCHUNK)],
                                 buf, sem)

        for i in range(_BLK // 16):
            ones[pl.ds(i * 16, 16)] = jnp.full((16,), 1.0, jnp.float32)

        pltpu.sync_copy(zeros_hbm.at[pl.ds(sid * rpt, rpt)], zb)
        pltpu.sync_copy(zb, acc.at[pl.ds(sid * rpt, rpt)])
        plsc.subcore_barrier()

        load_idx(base, idx.at[0], semI)
        pltpu.make_async_copy(main_hbm.at[0, pl.ds(0, _SCHUNK)],
                              idx.at[0], semI).wait()
        if wpt > 1:
            load_idx(base + _SCHUNK, idx.at[1], semI)

        def wave(w, _):
            prv = (w + ns - 1) % ns
            cur = w % ns
            nxt = (w + 1) % ns
            nn = (w + 2) % ns

            @pl.when(w > 0)
            def _():
                pltpu.make_async_copy(main_hbm.at[0, pl.ds(0, _SCHUNK)],
                                      idx.at[prv], semS).wait()

            for j in range(_SCHUNK):
                pltpu.async_copy(ones, acc.at[idx.at[cur, j]], semS,
                                 add=True)

            @pl.when(w < wpt - 1)
            def _():
                pltpu.make_async_copy(main_hbm.at[0, pl.ds(0, _SCHUNK)],
                                      idx.at[nxt], semI).wait()

            @pl.when(w < wpt - 2)
            def _():
                load_idx(base + (w + 2) * _SCHUNK, idx.at[nn], semI)

            return ()

        lax.fori_loop(0, wpt, wave, (), unroll=False)

        pltpu.make_async_copy(main_hbm.at[0, pl.ds(0, _SCHUNK)],
                              idx.at[(wpt - 1) % ns], semS).wait()
        plsc.subcore_barrier()
        pltpu.sync_copy(acc.at[pl.ds(sid * rpt, rpt)], zb)
        pltpu.sync_copy(zb, out_hbm.at[pl.ds(cid * n_pad + sid * rpt, rpt)])

    return pl.kernel(
        body,
        out_type=jax.ShapeDtypeStruct((_NC * n_pad,), jnp.float32),
        mesh=_mesh(),
        compiler_params=_SC_PARAMS,
        scratch_types=[
            pltpu.VMEM_SHARED((n_pad,), jnp.float32),
            pltpu.VMEM((_NSETS, _SCHUNK, _BLK), jnp.int32),
            pltpu.VMEM((_BLK,), jnp.float32),
            pltpu.VMEM((n_pad // _NS,), jnp.float32),
            pltpu.SemaphoreType.DMA,
            pltpu.SemaphoreType.DMA,
        ],
    )


def _make_agg_kernel(n_pad, ebm, ebx, width):
    schunk = _SCHUNK if width <= 8 else _SCHUNK // 2
    ns = _NSETS
    bpt = (ebm + ebx) // _NW
    wpt = bpt // schunk
    rpt = n_pad // _NS
    stage = ns * schunk * _BLK

    def body(g_hbm, main_hbm, xsrc_hbm, xdst_hbm, zeros_hbm, dummy_hbm,
             out_hbm, acc, sidx, didx, rows, semG, semS, semI):
        cid = lax.axis_index("c")
        sid = lax.axis_index("s")
        wid = cid * _NS + sid
        base = wid * bpt

        def load_idx2(blk, sbuf, dbuf, sem):
            @pl.when(blk < ebm)
            def _():
                pltpu.async_copy(main_hbm.at[0, pl.ds(blk, schunk)], sbuf,
                                 sem)
                pltpu.async_copy(main_hbm.at[1, pl.ds(blk, schunk)], dbuf,
                                 sem)

            @pl.when(blk >= ebm)
            def _():
                pltpu.async_copy(xsrc_hbm.at[pl.ds(blk - ebm, schunk)],
                                 sbuf, sem)
                pltpu.async_copy(xdst_hbm.at[pl.ds(blk - ebm, schunk)],
                                 dbuf, sem)

        def rowbuf(b):
            return rows.at[pl.ds(b * _BLK, _BLK)]

        def rowset(s):
            return rows.at[pl.ds(s * schunk * _BLK, schunk * _BLK)]

        pos = 0
        while pos < rpt:
            sz = min(stage, rpt - pos)
            pltpu.sync_copy(zeros_hbm.at[pl.ds(sid * rpt + pos, sz)],
                            rows.at[pl.ds(0, sz)])
            pltpu.sync_copy(rows.at[pl.ds(0, sz)],
                            acc.at[pl.ds(sid * rpt + pos, sz)])
            pos += sz
        plsc.subcore_barrier()

        load_idx2(base, sidx.at[0], didx.at[0], semI)
        pltpu.make_async_copy(main_hbm.at[0, pl.ds(0, schunk)], sidx.at[0],
                              semI).wait()
        pltpu.make_async_copy(main_hbm.at[0, pl.ds(0, schunk)], didx.at[0],
                              semI).wait()
        if wpt > 1:
            load_idx2(base + schunk, sidx.at[1], didx.at[1], semI)
        for j in range(schunk):
            pltpu.async_copy(g_hbm.at[sidx.at[0, j]], rowbuf(j), semG)

        def wave(w, _):
            prv = (w + ns - 1) % ns
            cur = w % ns
            nxt = (w + 1) % ns
            nn = (w + 2) % ns

            @pl.when(w > 0)
            def _():
                pltpu.make_async_copy(dummy_hbm, rowset(prv), semS).wait()

            pltpu.make_async_copy(dummy_hbm, rowset(cur), semG).wait()
            for j in range(schunk):
                pltpu.async_copy(rowbuf(cur * schunk + j),
                                 acc.at[didx.at[cur, j]], semS, add=True)

            @pl.when(w < wpt - 1)
            def _():
                pltpu.make_async_copy(main_hbm.at[0, pl.ds(0, schunk)],
                                      sidx.at[nxt], semI).wait()
                pltpu.make_async_copy(main_hbm.at[0, pl.ds(0, schunk)],
                                      didx.at[nxt], semI).wait()

            @pl.when(w < wpt - 2)
            def _():
                load_idx2(base + (w + 2) * schunk, sidx.at[nn],
                          didx.at[nn], semI)

            @pl.when(w < wpt - 1)
            def _():
                for j in range(schunk):
                    pltpu.async_copy(g_hbm.at[sidx.at[nxt, j]],
                                     rowbuf(nxt * schunk + j), semG)

            return ()

        lax.fori_loop(0, wpt, wave, (), unroll=False)

        pltpu.make_async_copy(dummy_hbm, rowset((wpt - 1) % ns),
                              semS).wait()
        plsc.subcore_barrier()
        pos = 0
        while pos < rpt:
            sz = min(stage, rpt - pos)
            pltpu.sync_copy(acc.at[pl.ds(sid * rpt + pos, sz)],
                            rows.at[pl.ds(0, sz)])
            pltpu.sync_copy(rows.at[pl.ds(0, sz)],
                            out_hbm.at[pl.ds(cid * n_pad + sid * rpt + pos,
                                             sz)])
            pos += sz

    return pl.kernel(
        body,
        out_type=jax.ShapeDtypeStruct((_NC * n_pad, width), jnp.float32),
        mesh=_mesh(),
        compiler_params=_SC_PARAMS,
        scratch_types=[
            pltpu.VMEM_SHARED((n_pad, width), jnp.float32),
            pltpu.VMEM((ns, schunk, _BLK), jnp.int32),
            pltpu.VMEM((ns, schunk, _BLK), jnp.int32),
            pltpu.VMEM((ns * schunk * _BLK, width), jnp.float32),
            pltpu.SemaphoreType.DMA,
            pltpu.SemaphoreType.DMA,
            pltpu.SemaphoreType.DMA,
        ],
    )


def _lin1_body(xw_ref, p0_ref, p1_ref, w1bd_ref, e8_ref, g_ref):
    dis = lax.rsqrt(jnp.maximum(p0_ref[...] + p1_ref[...], 1.0))
    disg = jnp.dot(dis, e8_ref[...], preferred_element_type=jnp.float32)
    h = jnp.dot(xw_ref[...], w1bd_ref[...],
                preferred_element_type=jnp.float32)
    g_ref[...] = h * disg


def _lin2_body(p0w_ref, p1w_ref, p0d_ref, p1d_ref, w2bd_ref, e8a_ref,
               e8b_ref, b1g_ref, g2_ref):
    dis = lax.rsqrt(jnp.maximum(p0d_ref[...] + p1d_ref[...], 1.0))
    disg = jnp.dot(dis, e8a_ref[...], preferred_element_type=jnp.float32)
    s = p0w_ref[...] + p1w_ref[...]
    f = jnp.maximum(s * disg + b1g_ref[...], 0.0)
    z = jnp.dot(f, w2bd_ref[...], preferred_element_type=jnp.float32)
    g2_ref[...] = z * jnp.dot(dis, e8b_ref[...],
                              preferred_element_type=jnp.float32)


def _out_body(p0w_ref, p1w_ref, p0d_ref, p1d_ref, e16_ref, gsum_ref,
              swp_ref, b2g_ref, sel_ref, o_ref):
    dis = lax.rsqrt(jnp.maximum(p0d_ref[...] + p1d_ref[...], 1.0))
    disg = jnp.dot(dis, e16_ref[...], preferred_element_type=jnp.float32)
    o = (p0w_ref[...] + p1w_ref[...]) * disg + b2g_ref[...]
    m = jnp.maximum(o, jnp.dot(o, swp_ref[...],
                               preferred_element_type=jnp.float32))
    e = jnp.exp(o - m) * sel_ref[...]
    lse = jnp.log(jnp.dot(e, gsum_ref[...],
                          preferred_element_type=jnp.float32))
    o_ref[...] = o - m - lse


def kernel(x, edge_index, W1, b1, W2, b2):
    n, d_in = x.shape
    e = edge_index.shape[1]
    d_hid = W1.shape[1]
    d_out = W2.shape[1]
    f32 = jnp.float32

    n_pad = _ceil_to(n + 1, 2048)
    rw1 = n_pad * d_hid // 128
    rw2 = n_pad * 8 // 128
    rdeg = n_pad // 8

    em = e - e % (_BLK * _SCHUNK)
    ep = _ceil_to(e + n, _NW * _SCHUNK * _BLK)
    npad_e = ep - (e + n)
    ebm = em // _BLK
    ebx = (ep - em) // _BLK
    loop = jnp.arange(n, dtype=jnp.int32)
    pad_src = jnp.arange(npad_e, dtype=jnp.int32) % 1024
    pad_dst = n + jnp.arange(npad_e, dtype=jnp.int32) % (n_pad - n)
    main = edge_index[:, :em].reshape(2, ebm, _BLK)
    xsrc = jnp.concatenate([edge_index[0, em:], loop, pad_src]).reshape(
        ebx, _BLK)
    xdst = jnp.concatenate([edge_index[1, em:], loop, pad_dst]).reshape(
        ebx, _BLK)

    zeros1 = jnp.zeros((n_pad,), f32)
    zeros_h = jnp.zeros((n_pad, d_hid), f32)
    zeros_o = jnp.zeros((n_pad, 8), f32)
    dummy_h = jnp.zeros((_SCHUNK // 2 * _BLK, d_hid), f32)
    dummy_o = jnp.zeros((_SCHUNK * _BLK, 8), f32)

    W1bd = jax.scipy.linalg.block_diag(*([W1] * 8))
    W2p = jnp.concatenate([W2, jnp.zeros((d_hid, 8 - d_out), f32)], axis=1)
    W2bd = jax.scipy.linalg.block_diag(*([W2p] * 8))
    E8_128 = jnp.repeat(jnp.eye(8, dtype=f32), d_hid, axis=1)
    E8_64 = jnp.repeat(jnp.eye(8, dtype=f32), 8, axis=1)
    E16_128 = jnp.repeat(jnp.eye(16, dtype=f32), 8, axis=1)
    b1g = jnp.tile(b1, 8).reshape(1, 128)
    b2g = jnp.tile(jnp.concatenate([b2, jnp.zeros((8 - d_out,), f32)]),
                   16).reshape(1, 128)
    col = jnp.arange(128)
    sel = (col % 8 < d_out).astype(f32).reshape(1, 128)
    gsum = ((col[:, None] // 8 == col[None, :] // 8)
            & (col[:, None] % 8 < d_out)).astype(f32)
    swap_idx = jnp.where(col % 8 < d_out, col ^ 1, col)
    swp = (col[:, None] == swap_idx[None, :]).astype(f32)

    degp = _make_deg_kernel(n_pad, ebm, ebx)(main, xdst, zeros1)
    p0d8 = degp.reshape(2 * rdeg, 8)
    p0d16 = degp.reshape(rdeg, 16)

    xw = jnp.pad(x.reshape(n // 8, 8 * d_in),
                 ((0, (n_pad - n) // 8), (0, 0)))

    bw = n_pad // 8 // 8
    bf = n_pad // 16 // 8

    g1w = pl.pallas_call(
        _lin1_body,
        grid=(8,),
        in_specs=[
            pl.BlockSpec((bw, 8 * d_in), lambda i: (i, 0)),
            pl.BlockSpec((bw, 8), lambda i: (i, 0)),
            pl.BlockSpec((bw, 8), lambda i: (i + 8, 0)),
            pl.BlockSpec((8 * d_in, 128), lambda i: (0, 0)),
            pl.BlockSpec((8, 128), lambda i: (0, 0)),
        ],
        out_specs=pl.BlockSpec((bw, 128), lambda i: (i, 0)),
        out_shape=jax.ShapeDtypeStruct((n_pad // 8, 128), f32),
    )(xw, p0d8, p0d8, W1bd, E8_128)
    g1 = g1w.reshape(n_pad, d_hid)

    aggp1 = _make_agg_kernel(n_pad, ebm, ebx, d_hid)(
        g1, main, xsrc, xdst, zeros_h, dummy_h)
    ap1w = aggp1.reshape(2 * n_pad * d_hid // 128, 128)

    g2w = pl.pallas_call(
        _lin2_body,
        grid=(8,),
        in_specs=[
            pl.BlockSpec((bw, 128), lambda i: (i, 0)),
            pl.BlockSpec((bw, 128), lambda i: (i + 8, 0)),
            pl.BlockSpec((bw, 8), lambda i: (i, 0)),
            pl.BlockSpec((bw, 8), lambda i: (i + 8, 0)),
            pl.BlockSpec((128, 64), lambda i: (0, 0)),
            pl.BlockSpec((8, 128), lambda i: (0, 0)),
            pl.BlockSpec((8, 64), lambda i: (0, 0)),
            pl.BlockSpec((1, 128), lambda i: (0, 0)),
        ],
        out_specs=pl.BlockSpec((bw, 64), lambda i: (i, 0)),
        out_shape=jax.ShapeDtypeStruct((n_pad // 8, 64), f32),
    )(ap1w, ap1w, p0d8, p0d8, W2bd, E8_128, E8_64, b1g)
    g2 = g2w.reshape(n_pad, 8)

    aggp2 = _make_agg_kernel(n_pad, ebm, ebx, 8)(
        g2, main, xsrc, xdst, zeros_o, dummy_o)
    ap2w = aggp2.reshape(2 * n_pad * 8 // 128, 128)

    outw = pl.pallas_call(
        _out_body,
        grid=(8,),
        in_specs=[
            pl.BlockSpec((bf, 128), lambda i: (i, 0)),
            pl.BlockSpec((bf, 128), lambda i: (i + 8, 0)),
            pl.BlockSpec((bf, 16), lambda i: (i, 0)),
            pl.BlockSpec((bf, 16), lambda i: (i + 8, 0)),
            pl.BlockSpec((16, 128), lambda i: (0, 0)),
            pl.BlockSpec((128, 128), lambda i: (0, 0)),
            pl.BlockSpec((128, 128), lambda i: (0, 0)),
            pl.BlockSpec((1, 128), lambda i: (0, 0)),
            pl.BlockSpec((1, 128), lambda i: (0, 0)),
        ],
        out_specs=pl.BlockSpec((bf, 128), lambda i: (i, 0)),
        out_shape=jax.ShapeDtypeStruct((n_pad // 16, 128), f32),
    )(ap2w, ap2w, p0d16, p0d16, E16_128, gsum, swp, b2g, sel)

    flat = outw.reshape(n_pad * 8)
    cols = [lax.slice(flat, (c,), (n * 8,), (8,)) for c in range(d_out)]
    return jnp.stack(cols, axis=1)
